# baseline (speedup 1.0000x reference)
import jax, jax.numpy as jnp
from jax.experimental import pallas as pl

_N, _BLK = 100000, 10000

def _body(x_ref, g_ref):
    g_ref[...] = jnp.zeros((_BLK, 3), jnp.float32)

def kernel(xyz):
    g = pl.pallas_call(
        _body,
        grid=(_N // _BLK,),
        in_specs=[pl.BlockSpec((8, 3), lambda i: (0, 0))],
        out_specs=pl.BlockSpec((_BLK, 3), lambda i: (i, 0)),
        out_shape=jax.ShapeDtypeStruct((_N, 3), jnp.float32),
    )(xyz)
    return g
